# baseline (speedup 1.0000x reference)
import jax
import jax.numpy as jnp
from jax.experimental import pallas as pl
from jax.experimental.pallas import tpu as pltpu

N, NFEAT, NHID, NCLASS = 8192, 256, 64, 16

BLK = 256
NBLK = N // BLK
RESB = 3
NSLOT = 3
KSPL = N // 2


def _ksplit_dot(a, h):
    return (jnp.dot(a[:, :KSPL], h[:KSPL, :],
                    preferred_element_type=jnp.float32)
            + jnp.dot(a[:, KSPL:], h[KSPL:, :],
                      preferred_element_type=jnp.float32))


def _h1_kernel(x_ref, W1_ref, h1_ref):
    h1_ref[...] = jnp.dot(x_ref[...], W1_ref[...],
                          preferred_element_type=jnp.float32
                          ).astype(jnp.bfloat16)


def _gcn_kernel(adj_ref, h1_ref, W2_ref, W3_ref, b1_ref, b2_ref, b3_ref,
                linW_ref, linb_ref,
                adjb_hbm, out_ref,
                res_ref, rbuf_ref, x1_ref, hcur_ref, h3_ref,
                acc1_ref, acc2_ref, acc3_ref, sems):
    i = pl.program_id(0)
    p = jax.lax.div(i, NBLK)
    j = jax.lax.rem(i, NBLK)

    def write_copy(b, slot):
        return pltpu.make_async_copy(
            rbuf_ref.at[slot],
            adjb_hbm.at[pl.ds(b * BLK, BLK), :],
            sems.at[slot])

    def read_copy(b, slot):
        return pltpu.make_async_copy(
            adjb_hbm.at[pl.ds(b * BLK, BLK), :],
            rbuf_ref.at[slot],
            sems.at[slot])

    def blockmax(acc_ref, v):
        @pl.when(j == 0)
        def _():
            acc_ref[...] = v

        @pl.when(j != 0)
        def _():
            acc_ref[...] = jnp.maximum(acc_ref[...], v)

    @pl.when(p == 0)
    def _():
        ab = adj_ref[...].astype(jnp.bfloat16)

        @pl.when(j < RESB)
        def _():
            res_ref[pl.ds(j * BLK, BLK), :] = ab

        @pl.when(j >= RESB)
        def _():
            slot = jax.lax.rem(j, NSLOT)

            @pl.when(j - NSLOT >= RESB)
            def _():
                write_copy(j - NSLOT, slot).wait()

            rbuf_ref[pl.ds(slot, 1)] = ab[None]
            write_copy(j, slot).start()

        y = _ksplit_dot(ab, h1_ref[...])
        yr = jnp.maximum(y + b1_ref[...], 0.0)
        x1_ref[pl.ds(j * BLK, BLK), :] = yr.astype(jnp.bfloat16)
        blockmax(acc1_ref, yr)

    @pl.when(p > 0)
    def _():
        @pl.when(i == NBLK)
        def _():
            hcur_ref[...] = jnp.dot(x1_ref[...], W2_ref[...],
                                    preferred_element_type=jnp.float32
                                    ).astype(jnp.bfloat16)

        @pl.when(i == 2 * NBLK)
        def _():
            hcur_ref[...] = h3_ref[...]

        b = j + 2
        @pl.when((b >= RESB) & (b < NBLK))
        def _():
            slot_b = jax.lax.rem(b, NSLOT)

            @pl.when((p == 1) & (b < RESB + NSLOT))
            def _():
                last_w = NBLK - 1 - jax.lax.rem(NBLK - 1 - slot_b, NSLOT)
                write_copy(last_w, slot_b).wait()

            read_copy(b, slot_b).start()

        def compute(abj):
            y = _ksplit_dot(abj, hcur_ref[...])
            yb2 = jnp.maximum(y + b2_ref[...], 0.0)

            @pl.when(p == 1)
            def _():
                h3_ref[pl.ds(j * BLK, BLK), :] = jnp.dot(
                    yb2, W3_ref[...],
                    preferred_element_type=jnp.float32).astype(jnp.bfloat16)
                blockmax(acc2_ref, yb2)

            @pl.when(p == 2)
            def _():
                blockmax(acc3_ref, y + b3_ref[...])

        @pl.when(j < RESB)
        def _():
            compute(res_ref[pl.ds(j * BLK, BLK), :])

        @pl.when(j >= RESB)
        def _():
            slot = jax.lax.rem(j, NSLOT)
            read_copy(j, slot).wait()
            compute(rbuf_ref[pl.ds(slot, 1)][0])

    @pl.when(i == 3 * NBLK - 1)
    def _():
        o1 = jnp.max(acc1_ref[...], axis=0, keepdims=True)
        o2 = jnp.max(acc2_ref[...], axis=0, keepdims=True)
        o3 = jnp.max(acc3_ref[...], axis=0, keepdims=True)
        logits = (jnp.sum(linW_ref[:, 0:NHID] * o1, axis=1)
                  + jnp.sum(linW_ref[:, NHID:2 * NHID] * o2, axis=1)
                  + jnp.sum(linW_ref[:, 2 * NHID:] * o3, axis=1)
                  + linb_ref[0, :])
        z = logits - jnp.max(logits)
        out_ref[0, :] = z - jnp.log(jnp.sum(jnp.exp(z)))


def kernel(x, adj, W1, b1, W2, b2, W3, b3, linW, linb):
    full = lambda shape: pl.BlockSpec(shape, lambda i: (0, 0))

    h1 = pl.pallas_call(
        _h1_kernel,
        out_shape=jax.ShapeDtypeStruct((N, NHID), jnp.bfloat16),
    )(x, W1)

    _, out = pl.pallas_call(
        _gcn_kernel,
        grid=(3 * NBLK,),
        in_specs=[
            pl.BlockSpec((BLK, N), lambda i: (jnp.minimum(i, NBLK - 1), 0)),
            full((N, NHID)),
            full((NHID, NHID)),
            full((NHID, NHID)),
            full((1, NHID)),
            full((1, NHID)),
            full((1, NHID)),
            full((NCLASS, 3 * NHID)),
            full((1, NCLASS)),
        ],
        out_specs=[
            pl.BlockSpec(memory_space=pltpu.MemorySpace.HBM),
            pl.BlockSpec((1, NCLASS), lambda i: (0, 0)),
        ],
        out_shape=[
            jax.ShapeDtypeStruct((N, N), jnp.bfloat16),
            jax.ShapeDtypeStruct((1, NCLASS), jnp.float32),
        ],
        scratch_shapes=[
            pltpu.VMEM((RESB * BLK, N), jnp.bfloat16),
            pltpu.VMEM((NSLOT, BLK, N), jnp.bfloat16),
            pltpu.VMEM((N, NHID), jnp.bfloat16),
            pltpu.VMEM((N, NHID), jnp.bfloat16),
            pltpu.VMEM((N, NHID), jnp.bfloat16),
            pltpu.VMEM((BLK, NHID), jnp.float32),
            pltpu.VMEM((BLK, NHID), jnp.float32),
            pltpu.VMEM((BLK, NHID), jnp.float32),
            pltpu.SemaphoreType.DMA((NSLOT,)),
        ],
        compiler_params=pltpu.CompilerParams(
            dimension_semantics=("arbitrary",)),
    )(adj, h1, W2, W3, b1.reshape(1, -1), b2.reshape(1, -1),
      b3.reshape(1, -1), linW, linb.reshape(1, -1))
    return out.reshape(NCLASS)
